# Initial kernel scaffold; baseline (speedup 1.0000x reference)
import jax
import jax.numpy as jnp
from jax.experimental import pallas as pl


def kernel(h_t, bow_mask, W_plus, b_plus, W_minus, b_minus):
    raise NotImplementedError("write your pallas kernel here")


# speedup vs baseline: 6.0382x; 6.0382x over previous
import jax
import jax.numpy as jnp
from jax import lax
from jax.experimental import pallas as pl
from jax.experimental.pallas import tpu as pltpu
from jax.experimental.pallas import tpu_sc as plsc

_B, _D, _V, _K = 32, 768, 100000, 256
_BV = 2048


def _dense_body(h_ref, wp_ref, bp_ref, wm_ref, bm_ref, mask_ref, up_ref, um_ref):
    h = h_ref[...]
    m = mask_ref[...]

    def one(w_ref, b_ref):
        z = jax.lax.dot_general(h, w_ref[...], (((1,), (1,)), ((), ())),
                                preferred_element_type=jnp.float32)
        z = z + b_ref[...]
        u = jnp.maximum(z, 0.0) + jnp.log1p(jnp.exp(-jnp.abs(z)))
        return u * m

    up_ref[...] = one(wp_ref, bp_ref)
    um_ref[...] = one(wm_ref, bm_ref)


def _dense(h_t, bow_mask, W_plus, b_plus, W_minus, b_minus):
    nb = (_V + _BV - 1) // _BV
    return pl.pallas_call(
        _dense_body,
        grid=(nb,),
        in_specs=[
            pl.BlockSpec((_B, _D), lambda j: (0, 0)),
            pl.BlockSpec((_BV, _D), lambda j: (j, 0)),
            pl.BlockSpec((1, _BV), lambda j: (0, j)),
            pl.BlockSpec((_BV, _D), lambda j: (j, 0)),
            pl.BlockSpec((1, _BV), lambda j: (0, j)),
            pl.BlockSpec((_B, _BV), lambda j: (0, j)),
        ],
        out_specs=[
            pl.BlockSpec((_B, _BV), lambda j: (0, j)),
            pl.BlockSpec((_B, _BV), lambda j: (0, j)),
        ],
        out_shape=[
            jax.ShapeDtypeStruct((_B, _V), jnp.float32),
            jax.ShapeDtypeStruct((_B, _V), jnp.float32),
        ],
    )(h_t, W_plus, b_plus.reshape(1, _V), W_minus, b_minus.reshape(1, _V), bow_mask)


_NV = _V // 16
_CHUNK = 10
_NCH = _NV // _CHUNK
_CB = 65
_CAP = _CB * 16
_TRIG = _CAP - 16
_KB = _K // 16
_NEG1 = -1.0


def _sel_body(up_hbm, um_hbm, op_hbm, om_hbm, row_v, cv, ci, kv, ki):
    c_ax = lax.axis_index("c")
    s_ax = lax.axis_index("s")
    w = s_ax * 2 + c_ax
    iota = lax.iota(jnp.int32, 16)

    def count_pass(thr, strict):
        def cb(b, acc):
            vi = plsc.bitcast(cv[pl.ds(b * 16, 16)], jnp.int32)
            cmp = (vi > thr) if strict else (vi >= thr)
            return acc + cmp.astype(jnp.int32)
        acc = lax.fori_loop(0, _CB, cb, jnp.zeros((16,), jnp.int32))
        return jnp.sum(acc)

    def compact():
        def bs(_, lh):
            lo, hi = lh
            mid = lo + (hi - lo) // 2
            big = count_pass(mid, False) >= _K
            return (jnp.where(big, mid, lo), jnp.where(big, hi, mid))
        lo, _hi = lax.fori_loop(
            0, 31, bs, (jnp.int32(0), jnp.int32(0x7FFFFFFF)))
        thr = lo
        r = _K - count_pass(thr, True)

        def cpb(b, carry):
            mo, ae = carry
            v = cv[pl.ds(b * 16, 16)]
            ix = ci[pl.ds(b * 16, 16)]
            vi = plsc.bitcast(v, jnp.int32)
            gt = vi > thr
            eq = vi == thr
            eqi = eq.astype(jnp.int32)
            excl = plsc.cumsum(eqi) - eqi
            keep = gt | (eq & ((excl + ae) < r))
            plsc.store_compressed(kv.at[pl.ds(mo, 16)], v, mask=keep)
            plsc.store_compressed(ki.at[pl.ds(mo, 16)], ix, mask=keep)
            return (mo + jnp.sum(keep.astype(jnp.int32)), ae + jnp.sum(eqi))
        lax.fori_loop(0, _CB, cpb, (jnp.int32(0), jnp.int32(0)))

        def cpy(b, z):
            cv[pl.ds(b * 16, 16)] = kv[pl.ds(b * 16, 16)]
            ci[pl.ds(b * 16, 16)] = ki[pl.ds(b * 16, 16)]
            return z
        lax.fori_loop(0, _KB, cpy, 0)

        def tl(b, z):
            cv[pl.ds(b * 16, 16)] = jnp.full((16,), _NEG1, jnp.float32)
            return z
        lax.fori_loop(_KB, _CB, tl, 0)
        return plsc.bitcast(jnp.broadcast_to(thr, (16,)), jnp.float32)

    def one_row(u_hbm, o_hbm):
        pltpu.sync_copy(u_hbm.at[w], row_v)

        def tl0(b, z):
            cv[pl.ds(b * 16, 16)] = jnp.full((16,), _NEG1, jnp.float32)
            return z
        lax.fori_loop(0, _CB, tl0, 0)

        def chunk(cix, carry):
            t, n = carry
            base = cix * (_CHUNK * 16)
            lm = row_v[pl.ds(base, 16)]
            for q in range(1, _CHUNK):
                lm = jnp.maximum(lm, row_v[pl.ds(base + q * 16, 16)])
            anyv = jnp.sum((lm > t).astype(jnp.int32))

            def rare(op):
                def vbody(q, op2):
                    t2, n2 = op2
                    v = row_v[pl.ds(base + q * 16, 16)]
                    m = v > t2
                    cnt = jnp.sum(m.astype(jnp.int32))

                    def app(op3):
                        t3, n3 = op3
                        iv = iota + (base + q * 16)
                        plsc.store_compressed(cv.at[pl.ds(n3, 16)], v, mask=m)
                        plsc.store_compressed(ci.at[pl.ds(n3, 16)], iv, mask=m)
                        n4 = n3 + cnt
                        return lax.cond(
                            n4 >= _TRIG,
                            lambda o: (compact(), jnp.int32(_K)),
                            lambda o: o, (t3, n4))
                    return lax.cond(cnt > 0, app, lambda o: o, (t2, n2))
                return lax.fori_loop(0, _CHUNK, vbody, op)
            return lax.cond(anyv > 0, rare, lambda o: o, (t, n))

        t0 = jnp.full((16,), _NEG1, jnp.float32)
        t, n = lax.fori_loop(0, _NCH, chunk, (t0, jnp.int32(0)))
        t, n = lax.cond(n > _K, lambda o: (compact(), jnp.int32(_K)),
                        lambda o: o, (t, n))

        def zf(i, z):
            row_v[pl.ds(i * 16, 16)] = jnp.zeros((16,), jnp.float32)
            return z
        lax.fori_loop(0, _NV, zf, 0)

        def sc(b, z):
            plsc.store_scatter(row_v, [ci[pl.ds(b * 16, 16)]],
                               cv[pl.ds(b * 16, 16)])
            return z
        lax.fori_loop(0, _KB, sc, 0)
        pltpu.sync_copy(row_v, o_hbm.at[w])

    one_row(up_hbm, op_hbm)
    one_row(um_hbm, om_hbm)


def _select(up, um):
    mesh = plsc.VectorSubcoreMesh(core_axis_name="c", subcore_axis_name="s",
                                  num_cores=2, num_subcores=16)
    f = pl.kernel(
        _sel_body,
        out_type=[
            jax.ShapeDtypeStruct((_B, _V), jnp.float32),
            jax.ShapeDtypeStruct((_B, _V), jnp.float32),
        ],
        mesh=mesh,
        compiler_params=pltpu.CompilerParams(needs_layout_passes=False),
        scratch_types=[
            pltpu.VMEM((_V,), jnp.float32),
            pltpu.VMEM((_CAP,), jnp.float32),
            pltpu.VMEM((_CAP,), jnp.int32),
            pltpu.VMEM((_K + 16,), jnp.float32),
            pltpu.VMEM((_K + 16,), jnp.int32),
        ],
    )
    return f(up, um)


def kernel(h_t, bow_mask, W_plus, b_plus, W_minus, b_minus):
    up, um = _dense(h_t, bow_mask, W_plus, b_plus, W_minus, b_minus)
    dsp, dsm = _select(up, um)
    return dsp, dsm


# speedup vs baseline: 8.9185x; 1.4770x over previous
import jax
import jax.numpy as jnp
from jax import lax
from jax.experimental import pallas as pl
from jax.experimental.pallas import tpu as pltpu
from jax.experimental.pallas import tpu_sc as plsc

_B, _D, _V, _K = 32, 768, 100000, 256
_BV = 2048


def _dense_body(h_ref, wp_ref, bp_ref, wm_ref, bm_ref, mask_ref, up_ref, um_ref):
    h = h_ref[...]
    m = mask_ref[...]

    def one(w_ref, b_ref):
        z = jax.lax.dot_general(h, w_ref[...], (((1,), (1,)), ((), ())),
                                preferred_element_type=jnp.float32)
        z = z + b_ref[...]
        u = jnp.maximum(z, 0.0) + jnp.log1p(jnp.exp(-jnp.abs(z)))
        return u * m

    up_ref[...] = one(wp_ref, bp_ref)
    um_ref[...] = one(wm_ref, bm_ref)


def _dense(h_t, bow_mask, W_plus, b_plus, W_minus, b_minus):
    nb = (_V + _BV - 1) // _BV
    return pl.pallas_call(
        _dense_body,
        grid=(nb,),
        in_specs=[
            pl.BlockSpec((_B, _D), lambda j: (0, 0)),
            pl.BlockSpec((_BV, _D), lambda j: (j, 0)),
            pl.BlockSpec((1, _BV), lambda j: (0, j)),
            pl.BlockSpec((_BV, _D), lambda j: (j, 0)),
            pl.BlockSpec((1, _BV), lambda j: (0, j)),
            pl.BlockSpec((_B, _BV), lambda j: (0, j)),
        ],
        out_specs=[
            pl.BlockSpec((_B, _BV), lambda j: (0, j)),
            pl.BlockSpec((_B, _BV), lambda j: (0, j)),
        ],
        out_shape=[
            jax.ShapeDtypeStruct((_B, _V), jnp.float32),
            jax.ShapeDtypeStruct((_B, _V), jnp.float32),
        ],
    )(h_t, W_plus, b_plus.reshape(1, _V), W_minus, b_minus.reshape(1, _V), bow_mask)


_NV = _V // 16
_CHUNK = 10
_NCH = _NV // _CHUNK
_CB = 65
_CAP = _CB * 16
_TRIG = _CAP - _CHUNK * 16
_KB = _K // 16
_NEG1 = -1.0


def _popcnt(mask):
    return plsc.all_reduce_population_count(mask)[0]


def _sel_body(up_hbm, um_hbm, op_hbm, om_hbm, row_v, cv, ci, kv, ki):
    c_ax = lax.axis_index("c")
    s_ax = lax.axis_index("s")
    w = s_ax * 2 + c_ax
    iota = lax.iota(jnp.int32, 16)

    def count_pass(thr, strict):
        def cb(b, acc):
            vi = plsc.bitcast(cv[pl.ds(b * 16, 16)], jnp.int32)
            cmp = (vi > thr) if strict else (vi >= thr)
            return acc + cmp.astype(jnp.int32)
        acc = lax.fori_loop(0, _CB, cb, jnp.zeros((16,), jnp.int32))
        return jnp.sum(acc)

    def compact():
        def bs(_, lh):
            lo, hi = lh
            mid = lo + (hi - lo) // 2
            big = count_pass(mid, False) >= _K
            return (jnp.where(big, mid, lo), jnp.where(big, hi, mid))
        lo, _hi = lax.fori_loop(
            0, 31, bs, (jnp.int32(0), jnp.int32(0x7FFFFFFF)))
        thr = lo
        r = _K - count_pass(thr, True)

        def cpb(b, carry):
            mo, ae = carry
            v = cv[pl.ds(b * 16, 16)]
            ix = ci[pl.ds(b * 16, 16)]
            vi = plsc.bitcast(v, jnp.int32)
            gt = vi > thr
            eq = vi == thr
            eqi = eq.astype(jnp.int32)
            excl = plsc.cumsum(eqi) - eqi
            keep = gt | (eq & ((excl + ae) < r))
            plsc.store_compressed(kv.at[pl.ds(mo, 16)], v, mask=keep)
            plsc.store_compressed(ki.at[pl.ds(mo, 16)], ix, mask=keep)
            return (mo + _popcnt(keep), ae + _popcnt(eq))
        lax.fori_loop(0, _CB, cpb, (jnp.int32(0), jnp.int32(0)))

        def cpy(b, z):
            cv[pl.ds(b * 16, 16)] = kv[pl.ds(b * 16, 16)]
            ci[pl.ds(b * 16, 16)] = ki[pl.ds(b * 16, 16)]
            return z
        lax.fori_loop(0, _KB, cpy, 0)

        def tl(b, z):
            cv[pl.ds(b * 16, 16)] = jnp.full((16,), _NEG1, jnp.float32)
            return z
        lax.fori_loop(_KB, _CB, tl, 0)
        return plsc.bitcast(jnp.broadcast_to(thr, (16,)), jnp.float32)

    def one_row(u_hbm, o_hbm):
        pltpu.sync_copy(u_hbm.at[w], row_v)

        def tl0(b, z):
            cv[pl.ds(b * 16, 16)] = jnp.full((16,), _NEG1, jnp.float32)
            return z
        lax.fori_loop(0, _CB, tl0, 0)

        def chunk(cix, carry):
            t, n = carry
            base = cix * (_CHUNK * 16)
            lm = row_v[pl.ds(base, 16)]
            for q in range(1, _CHUNK):
                lm = jnp.maximum(lm, row_v[pl.ds(base + q * 16, 16)])
            anyv = _popcnt(lm > t)

            def rare(op):
                t2, n2 = op
                for q in range(_CHUNK):
                    v = row_v[pl.ds(base + q * 16, 16)]
                    m = v > t2
                    iv = iota + (base + q * 16)
                    plsc.store_compressed(cv.at[pl.ds(n2, 16)], v, mask=m)
                    plsc.store_compressed(ci.at[pl.ds(n2, 16)], iv, mask=m)
                    n2 = n2 + _popcnt(m)
                return lax.cond(
                    n2 >= _TRIG,
                    lambda o: (compact(), jnp.int32(_K)),
                    lambda o: o, (t2, n2))
            return lax.cond(anyv > 0, rare, lambda o: o, (t, n))

        t0 = jnp.full((16,), _NEG1, jnp.float32)
        t, n = lax.fori_loop(0, _NCH, chunk, (t0, jnp.int32(0)))
        t, n = lax.cond(n > _K, lambda o: (compact(), jnp.int32(_K)),
                        lambda o: o, (t, n))

        def zf(i, z):
            b = i * (_CHUNK * 16)
            for q in range(_CHUNK):
                row_v[pl.ds(b + q * 16, 16)] = jnp.zeros((16,), jnp.float32)
            return z
        lax.fori_loop(0, _NCH, zf, 0)

        def sc(b, z):
            plsc.store_scatter(row_v, [ci[pl.ds(b * 16, 16)]],
                               cv[pl.ds(b * 16, 16)])
            return z
        lax.fori_loop(0, _KB, sc, 0)
        pltpu.sync_copy(row_v, o_hbm.at[w])

    one_row(up_hbm, op_hbm)
    one_row(um_hbm, om_hbm)


def _select(up, um):
    mesh = plsc.VectorSubcoreMesh(core_axis_name="c", subcore_axis_name="s",
                                  num_cores=2, num_subcores=16)
    f = pl.kernel(
        _sel_body,
        out_type=[
            jax.ShapeDtypeStruct((_B, _V), jnp.float32),
            jax.ShapeDtypeStruct((_B, _V), jnp.float32),
        ],
        mesh=mesh,
        compiler_params=pltpu.CompilerParams(needs_layout_passes=False),
        scratch_types=[
            pltpu.VMEM((_V,), jnp.float32),
            pltpu.VMEM((_CAP,), jnp.float32),
            pltpu.VMEM((_CAP,), jnp.int32),
            pltpu.VMEM((_K + 16,), jnp.float32),
            pltpu.VMEM((_K + 16,), jnp.int32),
        ],
    )
    return f(up, um)


def kernel(h_t, bow_mask, W_plus, b_plus, W_minus, b_minus):
    up, um = _dense(h_t, bow_mask, W_plus, b_plus, W_minus, b_minus)
    dsp, dsm = _select(up, um)
    return dsp, dsm


# speedup vs baseline: 10.7371x; 1.2039x over previous
import jax
import jax.numpy as jnp
from jax import lax
from jax.experimental import pallas as pl
from jax.experimental.pallas import tpu as pltpu
from jax.experimental.pallas import tpu_sc as plsc

_B, _D, _V, _K = 32, 768, 100000, 256
_BV = 2048


def _dense_body(h_ref, w_ref, b_ref, mask_ref, u_ref):
    z = jax.lax.dot_general(h_ref[...], w_ref[...], (((1,), (1,)), ((), ())),
                            preferred_element_type=jnp.float32)
    z = z + b_ref[...]
    u = jnp.maximum(z, 0.0) + jnp.log1p(jnp.exp(-jnp.abs(z)))
    u_ref[...] = u * mask_ref[...]


def _dense(h_t, bow_mask, W, b):
    nb = (_V + _BV - 1) // _BV
    return pl.pallas_call(
        _dense_body,
        grid=(nb,),
        in_specs=[
            pl.BlockSpec((_B, _D), lambda j: (0, 0)),
            pl.BlockSpec((_BV, _D), lambda j: (j, 0)),
            pl.BlockSpec((1, _BV), lambda j: (0, j)),
            pl.BlockSpec((_B, _BV), lambda j: (0, j)),
        ],
        out_specs=pl.BlockSpec((_B, _BV), lambda j: (0, j)),
        out_shape=jax.ShapeDtypeStruct((_B, _V), jnp.float32),
    )(h_t, W, b.reshape(1, _V), bow_mask)


_NV = _V // 16
_CHUNK = 10
_NCH = _NV // _CHUNK
_CB = 65
_CAP = _CB * 16
_TRIG = _CAP - _CHUNK * 16
_KB = _K // 16
_NEG1 = -1.0


def _popcnt(mask):
    return plsc.all_reduce_population_count(mask)[0]


def _sel_body(u_hbm, o_hbm, row_v, cv, ci, kv, ki):
    c_ax = lax.axis_index("c")
    s_ax = lax.axis_index("s")
    w = s_ax * 2 + c_ax
    iota = lax.iota(jnp.int32, 16)

    def count_pass(thr, strict):
        def cb(b, acc):
            vi = plsc.bitcast(cv[pl.ds(b * 16, 16)], jnp.int32)
            cmp = (vi > thr) if strict else (vi >= thr)
            return acc + cmp.astype(jnp.int32)
        acc = lax.fori_loop(0, _CB, cb, jnp.zeros((16,), jnp.int32))
        return jnp.sum(acc)

    def compact():
        def bs(_, lh):
            lo, hi = lh
            mid = lo + (hi - lo) // 2
            big = count_pass(mid, False) >= _K
            return (jnp.where(big, mid, lo), jnp.where(big, hi, mid))
        lo, _hi = lax.fori_loop(
            0, 31, bs, (jnp.int32(0), jnp.int32(0x7FFFFFFF)))
        thr = lo
        r = _K - count_pass(thr, True)

        def cpb(b, carry):
            mo, ae = carry
            v = cv[pl.ds(b * 16, 16)]
            ix = ci[pl.ds(b * 16, 16)]
            vi = plsc.bitcast(v, jnp.int32)
            gt = vi > thr
            eq = vi == thr
            eqi = eq.astype(jnp.int32)
            excl = plsc.cumsum(eqi) - eqi
            keep = gt | (eq & ((excl + ae) < r))
            plsc.store_compressed(kv.at[pl.ds(mo, 16)], v, mask=keep)
            plsc.store_compressed(ki.at[pl.ds(mo, 16)], ix, mask=keep)
            return (mo + _popcnt(keep), ae + _popcnt(eq))
        lax.fori_loop(0, _CB, cpb, (jnp.int32(0), jnp.int32(0)))

        def cpy(b, z):
            cv[pl.ds(b * 16, 16)] = kv[pl.ds(b * 16, 16)]
            ci[pl.ds(b * 16, 16)] = ki[pl.ds(b * 16, 16)]
            return z
        lax.fori_loop(0, _KB, cpy, 0)

        def tl(b, z):
            cv[pl.ds(b * 16, 16)] = jnp.full((16,), _NEG1, jnp.float32)
            return z
        lax.fori_loop(_KB, _CB, tl, 0)
        return plsc.bitcast(jnp.broadcast_to(thr, (16,)), jnp.float32)

    pltpu.sync_copy(u_hbm.at[w], row_v)

    def tl0(b, z):
        cv[pl.ds(b * 16, 16)] = jnp.full((16,), _NEG1, jnp.float32)
        return z
    lax.fori_loop(0, _CB, tl0, 0)

    def chunk(cix, carry):
        t, n = carry
        base = cix * (_CHUNK * 16)
        lm = row_v[pl.ds(base, 16)]
        for q in range(1, _CHUNK):
            lm = jnp.maximum(lm, row_v[pl.ds(base + q * 16, 16)])
        anyv = _popcnt(lm > t)

        def rare(op):
            t2, n2 = op
            for q in range(_CHUNK):
                v = row_v[pl.ds(base + q * 16, 16)]
                m = v > t2
                iv = iota + (base + q * 16)
                plsc.store_compressed(cv.at[pl.ds(n2, 16)], v, mask=m)
                plsc.store_compressed(ci.at[pl.ds(n2, 16)], iv, mask=m)
                n2 = n2 + _popcnt(m)
            return lax.cond(
                n2 >= _TRIG,
                lambda o: (compact(), jnp.int32(_K)),
                lambda o: o, (t2, n2))
        return lax.cond(anyv > 0, rare, lambda o: o, (t, n))

    t0 = jnp.full((16,), _NEG1, jnp.float32)
    t, n = lax.fori_loop(0, _NCH, chunk, (t0, jnp.int32(0)))
    t, n = lax.cond(n > _K, lambda o: (compact(), jnp.int32(_K)),
                    lambda o: o, (t, n))

    def zf(i, z):
        b = i * (_CHUNK * 16)
        for q in range(_CHUNK):
            row_v[pl.ds(b + q * 16, 16)] = jnp.zeros((16,), jnp.float32)
        return z
    lax.fori_loop(0, _NCH, zf, 0)

    def sc(b, z):
        plsc.store_scatter(row_v, [ci[pl.ds(b * 16, 16)]],
                           cv[pl.ds(b * 16, 16)])
        return z
    lax.fori_loop(0, _KB, sc, 0)
    pltpu.sync_copy(row_v, o_hbm.at[w])


def _select(u):
    mesh = plsc.VectorSubcoreMesh(core_axis_name="c", subcore_axis_name="s",
                                  num_cores=2, num_subcores=16)
    f = pl.kernel(
        _sel_body,
        out_type=jax.ShapeDtypeStruct((_B, _V), jnp.float32),
        mesh=mesh,
        compiler_params=pltpu.CompilerParams(needs_layout_passes=False),
        scratch_types=[
            pltpu.VMEM((_V,), jnp.float32),
            pltpu.VMEM((_CAP,), jnp.float32),
            pltpu.VMEM((_CAP,), jnp.int32),
            pltpu.VMEM((_K + 16,), jnp.float32),
            pltpu.VMEM((_K + 16,), jnp.int32),
        ],
    )
    return f(u)


def kernel(h_t, bow_mask, W_plus, b_plus, W_minus, b_minus):
    up = _dense(h_t, bow_mask, W_plus, b_plus)
    dsp = _select(up)
    um = _dense(h_t, bow_mask, W_minus, b_minus)
    dsm = _select(um)
    return dsp, dsm
